# baseline (speedup 1.0000x reference)
import functools

import jax
import jax.numpy as jnp
from jax import lax
from jax.experimental import pallas as pl
from jax.experimental.pallas import tpu as pltpu
from jax.experimental.pallas import tpu_sc as plsc

_BATCH = 16384
_L = 16
_NC = 2
_NS = 16
_NW = _NC * _NS
_B_PER_W = _BATCH // _NW
_CHUNK = 128
_NCHUNK = _B_PER_W // _CHUNK

_mesh = plsc.VectorSubcoreMesh(core_axis_name="c", subcore_axis_name="s")


@functools.partial(
    pl.kernel,
    mesh=_mesh,
    out_type=jax.ShapeDtypeStruct((_NW * _L,), jnp.float32),
    scratch_types=[
        pltpu.VMEM((_B_PER_W,), jnp.int32),
        pltpu.VMEM((_B_PER_W,), jnp.float32),
        pltpu.VMEM((_B_PER_W,), jnp.float32),
        pltpu.VMEM((_B_PER_W,), jnp.float32),
        pltpu.VMEM((_L,), jnp.float32),
        pltpu.VMEM((_L,), jnp.float32),
        pltpu.SemaphoreType.DMA,
        pltpu.SemaphoreType.DMA,
        pltpu.SemaphoreType.DMA,
        pltpu.SemaphoreType.DMA,
        pltpu.SemaphoreType.DMA,
    ],
)
def _sc_gather_sum(idx_hbm, a_hbm, b_hbm, c_hbm, m_hbm, out_hbm,
                   idx_v, av, bv, cv, mv, pv,
                   sem0, sem1, sem2, sem3, sem4):
    cid = lax.axis_index("c")
    sid = lax.axis_index("s")
    wid = sid * _NC + cid
    sems = (sem0, sem1, sem2, sem3)

    m_copy = pltpu.async_copy(m_hbm, mv.at[pl.ds(0, 1)], sem4)

    idx_copies = [
        pltpu.async_copy(
            idx_hbm.at[pl.ds(wid * _B_PER_W + j * _CHUNK, _CHUNK)],
            idx_v.at[pl.ds(j * _CHUNK, _CHUNK)], sems[j])
        for j in range(_NCHUNK)
    ]

    copies = []
    for j in range(_NCHUNK):
        s = pl.ds(j * _CHUNK, _CHUNK)
        idx_copies[j].wait()
        copies.append((pltpu.async_copy(a_hbm.at[idx_v.at[s]], av.at[s], sems[j]),
                       pltpu.async_copy(b_hbm.at[idx_v.at[s]], bv.at[s], sems[j]),
                       pltpu.async_copy(c_hbm.at[idx_v.at[s]], cv.at[s], sems[j])))
    acc_ac = jnp.zeros((_L,), jnp.float32)
    acc_b = jnp.zeros((_L,), jnp.float32)
    for j in range(_NCHUNK):
        for cp in copies[j]:
            cp.wait()

        def body(i, accs):
            a_ac, a_b = accs
            s = pl.ds(j * _CHUNK + i * _L, _L)
            return a_ac + av[s] + cv[s], a_b + bv[s]

        acc_ac, acc_b = lax.fori_loop(0, _CHUNK // _L, body,
                                      (acc_ac, acc_b), unroll=2)

    m_copy.wait()
    m = mv[...][0]
    pv[...] = acc_ac * m + acc_b
    pltpu.sync_copy(pv, out_hbm.at[pl.ds(wid * _L, _L)])


def kernel(c_t_indices, M_prev, A, B, C):
    m1 = jnp.reshape(M_prev, (1,)).astype(jnp.float32)
    partials = _sc_gather_sum(c_t_indices.astype(jnp.int32), A, B, C, m1)
    return jnp.sum(partials)
